# Initial kernel scaffold; baseline (speedup 1.0000x reference)
import jax
import jax.numpy as jnp
from jax.experimental import pallas as pl


def kernel(position, hash_table):
    raise NotImplementedError("write your pallas kernel here")


# speedup vs baseline: 1.0256x; 1.0256x over previous
import functools

import jax
import jax.numpy as jnp
from jax import lax
from jax.experimental import pallas as pl
from jax.experimental.pallas import tpu as pltpu
from jax.experimental.pallas import tpu_sc as plsc

N_DIM = 3
N_ENTRIES = 4194304
N_FEATURE = 4
BATCH = 524288
GRID = 512.0
P1 = 19349663
P2 = 83492791
MASK = N_ENTRIES - 1

NC = 2
NS = 16
NW = NC * NS
L = 16

PER_W = BATCH // NW
CHUNK = 256
NCHUNK = PER_W // CHUNK
NGRP = CHUNK // L
NIDX = CHUNK * 8


def _body(pos_hbm, table_hbm, out_hbm, pos_v, idx_v, sub_v, w_v, rows_v, out_v, sem):
    wid = lax.axis_index("s") * NC + lax.axis_index("c")
    iota = lax.iota(jnp.int32, L)
    rep4 = lax.shift_right_logical(iota, jnp.int32(2))
    f4 = lax.bitwise_and(iota, jnp.int32(3))
    iota3 = iota * jnp.int32(3)

    def chunk_body(chunk, carry):
        base = (wid * NCHUNK + chunk) * CHUNK
        pltpu.sync_copy(pos_hbm.at[pl.ds(base * 3, CHUNK * 3)], pos_v)

        def hash_grp(grp, c):
            p0 = grp * L
            xs = []
            fracs = []
            for d in range(N_DIM):
                x = plsc.load_gather(pos_v, [iota3 + (p0 * 3 + d)])
                scaled = x * jnp.float32(GRID)
                li = scaled.astype(jnp.int32)
                fracs.append(scaled - li.astype(jnp.float32))
                xs.append(li)
            one = jnp.float32(1.0)
            c0 = (fracs[0], one - fracs[0])
            c1 = (fracs[1], one - fracs[1])
            c2 = (fracs[2], one - fracs[2])
            t0 = (xs[0], xs[0] + jnp.int32(1))
            m1 = xs[1] * jnp.int32(P1)
            t1 = (m1, m1 + jnp.int32(P1))
            m2 = xs[2] * jnp.int32(P2)
            t2 = (m2, m2 + jnp.int32(P2))
            sbase = grp * 128
            for j in range(8):
                b0, b1, b2 = (j >> 2) & 1, (j >> 1) & 1, j & 1
                h = lax.bitwise_and(
                    lax.bitwise_xor(lax.bitwise_xor(t0[b0], t1[b1]), t2[b2]),
                    jnp.int32(MASK),
                )
                idx_v[pl.ds(sbase + j * L, L)] = lax.shift_right_logical(
                    h, jnp.int32(2)
                )
                sub_v[pl.ds(sbase + j * L, L)] = lax.shift_left(
                    lax.bitwise_and(h, jnp.int32(3)), jnp.int32(2)
                )
                w_v[pl.ds(sbase + j * L, L)] = (c0[b0] * c1[b1]) * c2[b2]
            return c

        lax.fori_loop(jnp.int32(0), jnp.int32(NGRP), hash_grp, jnp.int32(0),
                      unroll=False)

        pltpu.make_async_copy(table_hbm.at[idx_v], rows_v, sem).start()
        pltpu.make_async_copy(table_hbm.at[idx_v], rows_v, sem).wait()

        def acc_grp(grp, c):
            sbase = grp * 128
            for k in range(4):
                r = jnp.int32(4 * k) + rep4
                acc = jnp.zeros((L,), jnp.float32)
                for j in range(8):
                    s = (sbase + jnp.int32(j * L)) + r
                    sub = plsc.load_gather(sub_v, [s])
                    v = plsc.load_gather(rows_v, [s, sub + f4])
                    wv = plsc.load_gather(w_v, [s])
                    acc = acc + v * wv
                out_v[pl.ds(grp * 64 + k * L, L)] = acc
            return c

        lax.fori_loop(jnp.int32(0), jnp.int32(NGRP), acc_grp, jnp.int32(0),
                      unroll=False)

        pltpu.sync_copy(out_v, out_hbm.at[pl.ds(base * 4, CHUNK * 4)])
        return carry

    lax.fori_loop(jnp.int32(0), jnp.int32(NCHUNK), chunk_body, jnp.int32(0),
                  unroll=False)


@jax.jit
def kernel(position, hash_table):
    mesh = plsc.VectorSubcoreMesh(
        core_axis_name="c", subcore_axis_name="s", num_cores=NC, num_subcores=NS
    )
    run = functools.partial(
        pl.kernel,
        out_type=jax.ShapeDtypeStruct((BATCH * N_FEATURE,), jnp.float32),
        mesh=mesh,
        compiler_params=pltpu.CompilerParams(
            needs_layout_passes=False, use_tc_tiling_on_sc=False
        ),
        scratch_types=[
            pltpu.VMEM((CHUNK * 3,), jnp.float32),
            pltpu.VMEM((NIDX,), jnp.int32),
            pltpu.VMEM((NIDX,), jnp.int32),
            pltpu.VMEM((NIDX,), jnp.float32),
            pltpu.VMEM((NIDX, 16), jnp.float32),
            pltpu.VMEM((CHUNK * N_FEATURE,), jnp.float32),
            pltpu.SemaphoreType.DMA,
        ],
    )(_body)
    out_flat = run(
        position.reshape(-1), hash_table.reshape(N_ENTRIES // 4, 4 * N_FEATURE)
    )
    return out_flat.reshape(BATCH, N_FEATURE)
